# baseline (speedup 1.0000x reference)
import functools

import jax
import jax.numpy as jnp
from jax import lax
from jax.experimental import pallas as pl
from jax.experimental.pallas import tpu as pltpu
from jax.experimental.pallas import tpu_sc as plsc

N = 30400
E = 577600
D = 128
H = 64
NC = 2
NS = 16
GROUP = 64
NG_TILE = 566
EPT = NG_TILE * GROUP
EPAD = EPT * NS
G = EPAD // GROUP
NPAIR = NG_TILE // 2
ROWS_TILE = 1904
ROWS_LAST = N - 15 * ROWS_TILE


def _softplus_body(w_ref, o_ref):
    o_ref[...] = jax.nn.softplus(w_ref[...])


def _sc_body(xh, combo, agg0, agg1,
             acc, rows0, rows1, idx0, idx1, dstS0, dstS1,
             si0, si1, sg0, sg1, ss0, ss1):
    c = lax.axis_index("c")
    s = lax.axis_index("s")

    rows = (rows0, rows1)
    idxb = (idx0, idx1)
    dstS = (dstS0, dstS1)
    sem_i = (si0, si1)
    sem_g = (sg0, sg1)
    sem_s = (ss0, ss1)

    zero = jnp.zeros((16,), jnp.float32)
    for e in range(GROUP):
        for q in range(4):
            rows0[e, pl.ds(16 * q, 16)] = zero
    r0 = s * ROWS_TILE
    nfull = jnp.where(s == 15, 28, 29)

    def zacc(i, _):
        pltpu.sync_copy(rows0, acc.at[pl.ds(r0 + i * GROUP, GROUP)])
        return 0
    lax.fori_loop(0, nfull, zacc, 0)
    pltpu.sync_copy(rows0.at[pl.ds(0, 48)],
                    acc.at[pl.ds(r0 + nfull * GROUP, 48)])
    plsc.subcore_barrier()

    g_base = s * NG_TILE

    def idx_start(g, b):
        pltpu.async_copy(combo.at[g_base + g], idxb[b], sem_i[b])

    def idx_wait(g, b):
        pltpu.make_async_copy(combo.at[g_base + g], idxb[b],
                              sem_i[b]).wait()

    def gather_start(b):
        pltpu.async_copy(xh.at[idxb[b].at[c]], rows[b], sem_g[b])

    def gather_wait(b):
        pltpu.make_async_copy(xh.at[idxb[b].at[c]], rows[b], sem_g[b]).wait()

    def scatter_start(b, k):
        pltpu.async_copy(rows[b].at[pl.ds(16 * k, 16)],
                         acc.at[dstS[b].at[k]], sem_s[b], add=True)

    def scatter_wait(b):
        for k in range(GROUP // 16):
            pltpu.make_async_copy(rows[b].at[pl.ds(16 * k, 16)],
                                  acc.at[dstS[b].at[k]], sem_s[b]).wait()

    idx_start(0, 0)
    idx_start(1, 1)
    idx_wait(0, 0)
    gather_start(0)

    def pair(t, _):
        for b in (0, 1):
            g = 2 * t + b
            if b == 0:
                @pl.when(t >= 1)
                def _():
                    scatter_wait(1)
                @pl.when(t >= 1)
                def _():
                    idx_wait(g + 1, 1)
                @pl.when(t == 0)
                def _():
                    pltpu.make_async_copy(combo.at[g_base + 1], idx1,
                                          si1).wait()
                gather_start(1)
            else:
                scatter_wait(0)

                @pl.when(t < NPAIR - 1)
                def _():
                    idx_wait(g + 1, 0)
                    gather_start(0)
            gather_wait(b)
            for k in range(GROUP // 16):
                sl16 = pl.ds(16 * k, 16)
                dstS[b][k] = idxb[b][2, sl16]
                wv = plsc.bitcast(idxb[b][3, sl16], jnp.float32)
                for t16 in range(16):
                    we = wv[t16]
                    e = 16 * k + t16
                    for q in range(4):
                        sl = pl.ds(16 * q, 16)
                        rows[b][e, sl] = rows[b][e, sl] * we
                scatter_start(b, k)
            @pl.when(t < NPAIR - 1)
            def _():
                idx_start(g + 2, b)
        return 0
    lax.fori_loop(0, NPAIR, pair, 0)

    scatter_wait(1)
    plsc.subcore_barrier()

    @pl.when(jnp.logical_and(c == 0, s < 15))
    def _():
        pltpu.sync_copy(acc.at[pl.ds(r0, ROWS_TILE)],
                        agg0.at[pl.ds(r0, ROWS_TILE)])

    @pl.when(jnp.logical_and(c == 1, s < 15))
    def _():
        pltpu.sync_copy(acc.at[pl.ds(r0, ROWS_TILE)],
                        agg1.at[pl.ds(r0, ROWS_TILE)])

    @pl.when(jnp.logical_and(c == 0, s == 15))
    def _():
        pltpu.sync_copy(acc.at[pl.ds(r0, ROWS_LAST)],
                        agg0.at[pl.ds(r0, ROWS_LAST)])

    @pl.when(jnp.logical_and(c == 1, s == 15))
    def _():
        pltpu.sync_copy(acc.at[pl.ds(r0, ROWS_LAST)],
                        agg1.at[pl.ds(r0, ROWS_LAST)])


@functools.cache
def _sc_agg():
    return pl.kernel(
        _sc_body,
        out_type=(jax.ShapeDtypeStruct((N, H), jnp.float32),
                  jax.ShapeDtypeStruct((N, H), jnp.float32)),
        mesh=plsc.VectorSubcoreMesh(core_axis_name="c", subcore_axis_name="s",
                                    num_cores=NC, num_subcores=NS),
        scratch_types=[
            pltpu.VMEM_SHARED((N, H), jnp.float32),
            pltpu.VMEM((GROUP, H), jnp.float32),
            pltpu.VMEM((GROUP, H), jnp.float32),
            pltpu.VMEM((4, GROUP), jnp.int32),
            pltpu.VMEM((4, GROUP), jnp.int32),
            pltpu.VMEM((GROUP // 16, 16), jnp.int32),
            pltpu.VMEM((GROUP // 16, 16), jnp.int32),
            pltpu.SemaphoreType.DMA,
            pltpu.SemaphoreType.DMA,
            pltpu.SemaphoreType.DMA,
            pltpu.SemaphoreType.DMA,
            pltpu.SemaphoreType.DMA,
            pltpu.SemaphoreType.DMA,
        ],
        compiler_params=pltpu.CompilerParams(use_tc_tiling_on_sc=False,
                                             needs_layout_passes=False),
    )


def _root_body(x_ref, wr_ref, b_ref, o_ref):
    o_ref[...] = jnp.dot(x_ref[...], wr_ref[...],
                         preferred_element_type=jnp.float32) + b_ref[...]


def _epi_body(r_ref, a0_ref, a1_ref, w0_ref, w1_ref, o_ref):
    acc = jnp.dot(a0_ref[...], w0_ref[...], preferred_element_type=jnp.float32)
    acc += jnp.dot(a1_ref[...], w1_ref[...], preferred_element_type=jnp.float32)
    o_ref[...] = acc + r_ref[...]


ROWS_BLK = 1520


def _root_part(x, wroott, b2):
    grid = N // ROWS_BLK
    return pl.pallas_call(
        _root_body,
        grid=(grid,),
        in_specs=[
            pl.BlockSpec((ROWS_BLK, D), lambda i: (i, 0)),
            pl.BlockSpec((D, D), lambda i: (0, 0)),
            pl.BlockSpec((1, D), lambda i: (0, 0)),
        ],
        out_specs=pl.BlockSpec((ROWS_BLK, D), lambda i: (i, 0)),
        out_shape=jax.ShapeDtypeStruct((N, D), jnp.float32),
    )(x, wroott, b2)


def _epilogue(root, agg0, agg1, w0t, w1t):
    grid = N // ROWS_BLK
    return pl.pallas_call(
        _epi_body,
        grid=(grid,),
        in_specs=[
            pl.BlockSpec((ROWS_BLK, D), lambda i: (i, 0)),
            pl.BlockSpec((ROWS_BLK, H), lambda i: (i, 0)),
            pl.BlockSpec((ROWS_BLK, H), lambda i: (i, 0)),
            pl.BlockSpec((H, D), lambda i: (0, 0)),
            pl.BlockSpec((H, D), lambda i: (0, 0)),
        ],
        out_specs=pl.BlockSpec((ROWS_BLK, D), lambda i: (i, 0)),
        out_shape=jax.ShapeDtypeStruct((N, D), jnp.float32),
    )(root, agg0, agg1, w0t, w1t)


@jax.jit
def kernel(x, edge_index, edge_w, W_rel, b_rel, W_root):
    wp = jnp.pad(edge_w, (0, 384 - edge_w.shape[0])).reshape(3, 128)
    w_sp = pl.pallas_call(
        _softplus_body,
        out_shape=jax.ShapeDtypeStruct((3, 128), jnp.float32),
    )(wp).reshape(-1)[:edge_w.shape[0]]

    n_graphs = N // 19
    w_full = jnp.tile(w_sp, n_graphs)
    w_bits = lax.bitcast_convert_type(
        jnp.pad(w_full, (0, EPAD - E)), jnp.int32)

    src = edge_index[0]
    dst = edge_index[1]
    srcp = jnp.pad(src, (0, EPAD - E))
    dstp = jnp.pad(dst, (0, EPAD - E))

    combo = jnp.stack([2 * srcp, 2 * srcp + 1, dstp, w_bits]) \
               .reshape(4, G, GROUP).transpose(1, 0, 2)

    xh = x.reshape(2 * N, H)

    agg0, agg1 = _sc_agg()(xh, combo)

    w0t = W_rel[:, :H].T
    w1t = W_rel[:, H:].T
    root = _root_part(x, W_root.T, b_rel.reshape(1, D))
    return _epilogue(root, agg0, agg1, w0t, w1t)


# speedup vs baseline: 1.0141x; 1.0141x over previous
import functools

import jax
import jax.numpy as jnp
from jax import lax
from jax.experimental import pallas as pl
from jax.experimental.pallas import tpu as pltpu
from jax.experimental.pallas import tpu_sc as plsc

N = 30400
E = 577600
D = 128
H = 64
NC = 2
NS = 16
GROUP = 64
NG_TILE = 566
EPT = NG_TILE * GROUP
EPAD = EPT * NS
G = EPAD // GROUP
NPAIR = NG_TILE // 2
ROWS_TILE = 1904
ROWS_LAST = N - 15 * ROWS_TILE


def _softplus_body(w_ref, o_ref):
    o_ref[...] = jax.nn.softplus(w_ref[...])


def _sc_body(xh, combo, agg0, agg1,
             acc, rows0, rows1, idx0, idx1, dstS0, dstS1,
             si0, si1, sg0, sg1, ss0, ss1):
    c = lax.axis_index("c")
    s = lax.axis_index("s")

    rows = (rows0, rows1)
    idxb = (idx0, idx1)
    dstS = (dstS0, dstS1)
    sem_i = (si0, si1)
    sem_g = (sg0, sg1)
    sem_s = (ss0, ss1)

    zero = jnp.zeros((16,), jnp.float32)
    for e in range(GROUP):
        for q in range(4):
            rows0[e, pl.ds(16 * q, 16)] = zero
    r0 = s * ROWS_TILE
    nfull = jnp.where(s == 15, 28, 29)

    def zacc(i, _):
        pltpu.sync_copy(rows0, acc.at[pl.ds(r0 + i * GROUP, GROUP)])
        return 0
    lax.fori_loop(0, nfull, zacc, 0)
    pltpu.sync_copy(rows0.at[pl.ds(0, 48)],
                    acc.at[pl.ds(r0 + nfull * GROUP, 48)])
    plsc.subcore_barrier()

    g_base = s * NG_TILE

    def idx_start(g, b):
        pltpu.async_copy(combo.at[g_base + g], idxb[b], sem_i[b])

    def idx_wait(g, b):
        pltpu.make_async_copy(combo.at[g_base + g], idxb[b],
                              sem_i[b]).wait()

    def gather_start(b):
        pltpu.async_copy(xh.at[idxb[b].at[c]], rows[b], sem_g[b])

    def gather_wait(b):
        pltpu.make_async_copy(xh.at[idxb[b].at[c]], rows[b], sem_g[b]).wait()

    def scatter_start(b, k):
        pltpu.async_copy(rows[b].at[pl.ds(16 * k, 16)],
                         acc.at[dstS[b].at[k]], sem_s[b], add=True)

    def scatter_wait(b):
        for k in range(GROUP // 16):
            pltpu.make_async_copy(rows[b].at[pl.ds(16 * k, 16)],
                                  acc.at[dstS[b].at[k]], sem_s[b]).wait()

    idx_start(0, 0)
    idx_start(1, 1)
    idx_wait(0, 0)
    gather_start(0)

    def pair(t, _):
        for b in (0, 1):
            g = 2 * t + b
            if b == 0:
                @pl.when(t >= 1)
                def _():
                    scatter_wait(1)
                @pl.when(t >= 1)
                def _():
                    idx_wait(g + 1, 1)
                @pl.when(t == 0)
                def _():
                    pltpu.make_async_copy(combo.at[g_base + 1], idx1,
                                          si1).wait()
                gather_start(1)
            else:
                scatter_wait(0)

                @pl.when(t < NPAIR - 1)
                def _():
                    idx_wait(g + 1, 0)
                    gather_start(0)
            gather_wait(b)
            for k in range(GROUP // 16):
                sl16 = pl.ds(16 * k, 16)
                dstS[b][k] = idxb[b][2, sl16]
                wv = plsc.bitcast(idxb[b][3, sl16], jnp.float32)
                for t16 in range(16):
                    we = wv[t16]
                    e = 16 * k + t16
                    for q in range(4):
                        sl = pl.ds(16 * q, 16)
                        rows[b][e, sl] = rows[b][e, sl] * we
                scatter_start(b, k)
            @pl.when(t < NPAIR - 1)
            def _():
                idx_start(g + 2, b)
        return 0
    lax.fori_loop(0, NPAIR, pair, 0)

    scatter_wait(1)
    plsc.subcore_barrier()

    @pl.when(jnp.logical_and(c == 0, s < 15))
    def _():
        pltpu.sync_copy(acc.at[pl.ds(r0, ROWS_TILE)],
                        agg0.at[pl.ds(r0, ROWS_TILE)])

    @pl.when(jnp.logical_and(c == 1, s < 15))
    def _():
        pltpu.sync_copy(acc.at[pl.ds(r0, ROWS_TILE)],
                        agg1.at[pl.ds(r0, ROWS_TILE)])

    @pl.when(jnp.logical_and(c == 0, s == 15))
    def _():
        pltpu.sync_copy(acc.at[pl.ds(r0, ROWS_LAST)],
                        agg0.at[pl.ds(r0, ROWS_LAST)])

    @pl.when(jnp.logical_and(c == 1, s == 15))
    def _():
        pltpu.sync_copy(acc.at[pl.ds(r0, ROWS_LAST)],
                        agg1.at[pl.ds(r0, ROWS_LAST)])


@functools.cache
def _sc_agg():
    return pl.kernel(
        _sc_body,
        out_type=(jax.ShapeDtypeStruct((N, H), jnp.float32),
                  jax.ShapeDtypeStruct((N, H), jnp.float32)),
        mesh=plsc.VectorSubcoreMesh(core_axis_name="c", subcore_axis_name="s",
                                    num_cores=NC, num_subcores=NS),
        scratch_types=[
            pltpu.VMEM_SHARED((N, H), jnp.float32),
            pltpu.VMEM((GROUP, H), jnp.float32),
            pltpu.VMEM((GROUP, H), jnp.float32),
            pltpu.VMEM((4, GROUP), jnp.int32),
            pltpu.VMEM((4, GROUP), jnp.int32),
            pltpu.VMEM((GROUP // 16, 16), jnp.int32),
            pltpu.VMEM((GROUP // 16, 16), jnp.int32),
            pltpu.SemaphoreType.DMA,
            pltpu.SemaphoreType.DMA,
            pltpu.SemaphoreType.DMA,
            pltpu.SemaphoreType.DMA,
            pltpu.SemaphoreType.DMA,
            pltpu.SemaphoreType.DMA,
        ],
        compiler_params=pltpu.CompilerParams(use_tc_tiling_on_sc=False,
                                             needs_layout_passes=False),
    )


def _epi_body(x_ref, a0_ref, a1_ref, wr_ref, w0_ref, w1_ref, b_ref, o_ref):
    acc = jnp.dot(x_ref[...], wr_ref[...], preferred_element_type=jnp.float32)
    acc += jnp.dot(a0_ref[...], w0_ref[...], preferred_element_type=jnp.float32)
    acc += jnp.dot(a1_ref[...], w1_ref[...], preferred_element_type=jnp.float32)
    o_ref[...] = acc + b_ref[...]


ROWS_BLK = 1520


def _epilogue(x, agg0, agg1, wroott, w0t, w1t, b2):
    grid = N // ROWS_BLK
    return pl.pallas_call(
        _epi_body,
        grid=(grid,),
        in_specs=[
            pl.BlockSpec((ROWS_BLK, D), lambda i: (i, 0)),
            pl.BlockSpec((ROWS_BLK, H), lambda i: (i, 0)),
            pl.BlockSpec((ROWS_BLK, H), lambda i: (i, 0)),
            pl.BlockSpec((D, D), lambda i: (0, 0)),
            pl.BlockSpec((H, D), lambda i: (0, 0)),
            pl.BlockSpec((H, D), lambda i: (0, 0)),
            pl.BlockSpec((1, D), lambda i: (0, 0)),
        ],
        out_specs=pl.BlockSpec((ROWS_BLK, D), lambda i: (i, 0)),
        out_shape=jax.ShapeDtypeStruct((N, D), jnp.float32),
    )(x, agg0, agg1, wroott, w0t, w1t, b2)


@jax.jit
def kernel(x, edge_index, edge_w, W_rel, b_rel, W_root):
    wp = jnp.pad(edge_w, (0, 384 - edge_w.shape[0])).reshape(3, 128)
    w_sp = pl.pallas_call(
        _softplus_body,
        out_shape=jax.ShapeDtypeStruct((3, 128), jnp.float32),
    )(wp).reshape(-1)[:edge_w.shape[0]]

    n_graphs = N // 19
    w_full = jnp.tile(w_sp, n_graphs)
    w_bits = lax.bitcast_convert_type(
        jnp.pad(w_full, (0, EPAD - E)), jnp.int32)

    src = edge_index[0]
    dst = edge_index[1]
    srcp = jnp.pad(src, (0, EPAD - E))
    dstp = jnp.pad(dst, (0, EPAD - E))

    srcp2 = (2 * srcp).reshape(G, GROUP)
    combo = jnp.stack([srcp2, srcp2 + 1, dstp.reshape(G, GROUP),
                       w_bits.reshape(G, GROUP)], axis=1)

    xh = x.reshape(2 * N, H)

    agg0, agg1 = _sc_agg()(xh, combo)

    w0t = W_rel[:, :H].T
    w1t = W_rel[:, H:].T
    return _epilogue(x, agg0, agg1, W_root.T, w0t, w1t, b_rel.reshape(1, D))


# speedup vs baseline: 1.0226x; 1.0084x over previous
import functools

import jax
import jax.numpy as jnp
from jax import lax
from jax.experimental import pallas as pl
from jax.experimental.pallas import tpu as pltpu
from jax.experimental.pallas import tpu_sc as plsc

N = 30400
E = 577600
D = 128
H = 64
NC = 2
NS = 16
GROUP = 64
NG_TILE = 566
EPT = NG_TILE * GROUP
EPAD = EPT * NS
G = EPAD // GROUP
NPAIR = NG_TILE // 2
ROWS_TILE = 1904
ROWS_LAST = N - 15 * ROWS_TILE


def _sc_body(xh, combo, agg0, agg1,
             acc, rows0, rows1, idx0, idx1, dstS0, dstS1,
             si0, si1, sg0, sg1, ss0, ss1):
    c = lax.axis_index("c")
    s = lax.axis_index("s")

    rows = (rows0, rows1)
    idxb = (idx0, idx1)
    dstS = (dstS0, dstS1)
    sem_i = (si0, si1)
    sem_g = (sg0, sg1)
    sem_s = (ss0, ss1)

    zero = jnp.zeros((16,), jnp.float32)
    for e in range(GROUP):
        for q in range(4):
            rows0[e, pl.ds(16 * q, 16)] = zero
    r0 = s * ROWS_TILE
    nfull = jnp.where(s == 15, 28, 29)

    def zacc(i, _):
        pltpu.sync_copy(rows0, acc.at[pl.ds(r0 + i * GROUP, GROUP)])
        return 0
    lax.fori_loop(0, nfull, zacc, 0)
    pltpu.sync_copy(rows0.at[pl.ds(0, 48)],
                    acc.at[pl.ds(r0 + nfull * GROUP, 48)])
    plsc.subcore_barrier()

    g_base = s * NG_TILE

    def idx_start(g, b):
        pltpu.async_copy(combo.at[g_base + g], idxb[b], sem_i[b])

    def idx_wait(g, b):
        pltpu.make_async_copy(combo.at[g_base + g], idxb[b],
                              sem_i[b]).wait()

    def gather_start(b):
        pltpu.async_copy(xh.at[idxb[b].at[c]], rows[b], sem_g[b])

    def gather_wait(b):
        pltpu.make_async_copy(xh.at[idxb[b].at[c]], rows[b], sem_g[b]).wait()

    def scatter_start(b, k):
        pltpu.async_copy(rows[b].at[pl.ds(16 * k, 16)],
                         acc.at[dstS[b].at[k]], sem_s[b], add=True)

    def scatter_wait(b):
        for k in range(GROUP // 16):
            pltpu.make_async_copy(rows[b].at[pl.ds(16 * k, 16)],
                                  acc.at[dstS[b].at[k]], sem_s[b]).wait()

    idx_start(0, 0)
    idx_start(1, 1)
    idx_wait(0, 0)
    gather_start(0)

    def pair(t, _):
        for b in (0, 1):
            g = 2 * t + b
            if b == 0:
                @pl.when(t >= 1)
                def _():
                    scatter_wait(1)
                @pl.when(t >= 1)
                def _():
                    idx_wait(g + 1, 1)
                @pl.when(t == 0)
                def _():
                    pltpu.make_async_copy(combo.at[g_base + 1], idx1,
                                          si1).wait()
                gather_start(1)
            else:
                scatter_wait(0)

                @pl.when(t < NPAIR - 1)
                def _():
                    idx_wait(g + 1, 0)
                    gather_start(0)
            gather_wait(b)
            for k in range(GROUP // 16):
                sl16 = pl.ds(16 * k, 16)
                dstS[b][k] = idxb[b][2, sl16]
                wv = plsc.bitcast(idxb[b][3, sl16], jnp.float32)
                for t16 in range(16):
                    we = wv[t16]
                    e = 16 * k + t16
                    for q in range(4):
                        sl = pl.ds(16 * q, 16)
                        rows[b][e, sl] = rows[b][e, sl] * we
                scatter_start(b, k)
            @pl.when(t < NPAIR - 1)
            def _():
                idx_start(g + 2, b)
        return 0
    lax.fori_loop(0, NPAIR, pair, 0)

    scatter_wait(1)
    plsc.subcore_barrier()

    @pl.when(jnp.logical_and(c == 0, s < 15))
    def _():
        pltpu.sync_copy(acc.at[pl.ds(r0, ROWS_TILE)],
                        agg0.at[pl.ds(r0, ROWS_TILE)])

    @pl.when(jnp.logical_and(c == 1, s < 15))
    def _():
        pltpu.sync_copy(acc.at[pl.ds(r0, ROWS_TILE)],
                        agg1.at[pl.ds(r0, ROWS_TILE)])

    @pl.when(jnp.logical_and(c == 0, s == 15))
    def _():
        pltpu.sync_copy(acc.at[pl.ds(r0, ROWS_LAST)],
                        agg0.at[pl.ds(r0, ROWS_LAST)])

    @pl.when(jnp.logical_and(c == 1, s == 15))
    def _():
        pltpu.sync_copy(acc.at[pl.ds(r0, ROWS_LAST)],
                        agg1.at[pl.ds(r0, ROWS_LAST)])


@functools.cache
def _sc_agg():
    return pl.kernel(
        _sc_body,
        out_type=(jax.ShapeDtypeStruct((N, H), jnp.float32),
                  jax.ShapeDtypeStruct((N, H), jnp.float32)),
        mesh=plsc.VectorSubcoreMesh(core_axis_name="c", subcore_axis_name="s",
                                    num_cores=NC, num_subcores=NS),
        scratch_types=[
            pltpu.VMEM_SHARED((N, H), jnp.float32),
            pltpu.VMEM((GROUP, H), jnp.float32),
            pltpu.VMEM((GROUP, H), jnp.float32),
            pltpu.VMEM((4, GROUP), jnp.int32),
            pltpu.VMEM((4, GROUP), jnp.int32),
            pltpu.VMEM((GROUP // 16, 16), jnp.int32),
            pltpu.VMEM((GROUP // 16, 16), jnp.int32),
            pltpu.SemaphoreType.DMA,
            pltpu.SemaphoreType.DMA,
            pltpu.SemaphoreType.DMA,
            pltpu.SemaphoreType.DMA,
            pltpu.SemaphoreType.DMA,
            pltpu.SemaphoreType.DMA,
        ],
        compiler_params=pltpu.CompilerParams(use_tc_tiling_on_sc=False,
                                             needs_layout_passes=False),
    )


def _epi_body(x_ref, a0_ref, a1_ref, wr_ref, wrel_ref, b_ref, o_ref):
    dn = (((1,), (1,)), ((), ()))
    acc = lax.dot_general(x_ref[...], wr_ref[...], dn,
                          preferred_element_type=jnp.float32)
    acc += lax.dot_general(a0_ref[...], wrel_ref[:, :H], dn,
                           preferred_element_type=jnp.float32)
    acc += lax.dot_general(a1_ref[...], wrel_ref[:, H:], dn,
                           preferred_element_type=jnp.float32)
    o_ref[...] = acc + b_ref[...]


ROWS_BLK = 1520


def _epilogue(x, agg0, agg1, wroot, wrel, b2):
    grid = N // ROWS_BLK
    return pl.pallas_call(
        _epi_body,
        grid=(grid,),
        in_specs=[
            pl.BlockSpec((ROWS_BLK, D), lambda i: (i, 0)),
            pl.BlockSpec((ROWS_BLK, H), lambda i: (i, 0)),
            pl.BlockSpec((ROWS_BLK, H), lambda i: (i, 0)),
            pl.BlockSpec((D, D), lambda i: (0, 0)),
            pl.BlockSpec((D, D), lambda i: (0, 0)),
            pl.BlockSpec((1, D), lambda i: (0, 0)),
        ],
        out_specs=pl.BlockSpec((ROWS_BLK, D), lambda i: (i, 0)),
        out_shape=jax.ShapeDtypeStruct((N, D), jnp.float32),
    )(x, agg0, agg1, wroot, wrel, b2)


G_BLK = 2264


def _combo_body(s_ref, d_ref, w_ref, o_ref):
    s2 = 2 * s_ref[...]
    o_ref[:, 0:GROUP] = s2
    o_ref[:, GROUP:2 * GROUP] = s2 + 1
    o_ref[:, 2 * GROUP:3 * GROUP] = d_ref[...]
    o_ref[:, 3 * GROUP:4 * GROUP] = lax.bitcast_convert_type(
        jax.nn.softplus(w_ref[...]), jnp.int32)


def _combo_build(srcp, dstp, wraw):
    grid = G // G_BLK
    return pl.pallas_call(
        _combo_body,
        grid=(grid,),
        in_specs=[
            pl.BlockSpec((G_BLK, GROUP), lambda i: (i, 0)),
            pl.BlockSpec((G_BLK, GROUP), lambda i: (i, 0)),
            pl.BlockSpec((G_BLK, GROUP), lambda i: (i, 0)),
        ],
        out_specs=pl.BlockSpec((G_BLK, 4 * GROUP), lambda i: (i, 0)),
        out_shape=jax.ShapeDtypeStruct((G, 4 * GROUP), jnp.int32),
    )(srcp.reshape(G, GROUP), dstp.reshape(G, GROUP), wraw.reshape(G, GROUP))


@jax.jit
def kernel(x, edge_index, edge_w, W_rel, b_rel, W_root):
    n_graphs = N // 19
    w_raw = jnp.pad(jnp.tile(edge_w, n_graphs), (0, EPAD - E),
                    constant_values=-1e9)

    src = edge_index[0]
    dst = edge_index[1]
    srcp = jnp.pad(src, (0, EPAD - E))
    dstp = jnp.pad(dst, (0, EPAD - E))

    combo = _combo_build(srcp, dstp, w_raw).reshape(G, 4, GROUP)

    xh = x.reshape(2 * N, H)

    agg0, agg1 = _sc_agg()(xh, combo)

    return _epilogue(x, agg0, agg1, W_root, W_rel, b_rel.reshape(1, D))
